# initial kernel scaffold (unmeasured)
import jax
import jax.numpy as jnp
from jax import lax
from jax.experimental import pallas as pl
from jax.experimental.pallas import tpu as pltpu

N_DEV = 4


def kernel(A, B):
    m, k = A.shape
    k2, n = B.shape

    def body(a_ref, b_ref, out_ref, comm_ref, send_sems, recv_sems):
        my = lax.axis_index("i")
        left = (my - 1) % N_DEV
        right = (my + 1) % N_DEV

        barrier_sem = pltpu.get_barrier_semaphore()
        for nbr in (left, right):
            pl.semaphore_signal(
                barrier_sem, inc=1,
                device_id=(nbr,), device_id_type=pl.DeviceIdType.MESH,
            )
        pl.semaphore_wait(barrier_sem, 2)

        partial = jnp.dot(a_ref[:, :], b_ref[:, :],
                          preferred_element_type=jnp.float32)
        comm_ref[0, :, :] = partial
        out_ref[:, :] = partial

        for h in range(N_DEV - 1):
            rdma = pltpu.make_async_remote_copy(
                src_ref=comm_ref.at[h],
                dst_ref=comm_ref.at[h + 1],
                send_sem=send_sems.at[h],
                recv_sem=recv_sems.at[h],
                device_id=(right,),
                device_id_type=pl.DeviceIdType.MESH,
            )
            rdma.start()
            rdma.wait()
            out_ref[:, :] += comm_ref[h + 1, :, :]

        z = out_ref[:, :]
        out_ref[:, :] = z * (1.0 / (1.0 + jnp.exp(-z)))

    return pl.pallas_call(
        body,
        out_shape=jax.ShapeDtypeStruct((m, n), jnp.float32),
        in_specs=[
            pl.BlockSpec(memory_space=pltpu.VMEM),
            pl.BlockSpec(memory_space=pltpu.VMEM),
        ],
        out_specs=pl.BlockSpec(memory_space=pltpu.VMEM),
        scratch_shapes=[
            pltpu.VMEM((N_DEV, m, n), jnp.float32),
            pltpu.SemaphoreType.DMA((N_DEV - 1,)),
            pltpu.SemaphoreType.DMA((N_DEV - 1,)),
        ],
        compiler_params=pltpu.CompilerParams(collective_id=0),
    )(A, B)


# baseline (device time: 335448 ns/iter reference)
import jax
import jax.numpy as jnp
from jax import lax
from jax.experimental import pallas as pl
from jax.experimental.pallas import tpu as pltpu

N_DEV = 4


def kernel(A, B):
    m, k = A.shape
    k2, n = B.shape

    def body(a_ref, b_ref, out_ref, comm_ref, send_sems, recv_sems):
        my = lax.axis_index("i")
        left = (my - 1) % N_DEV
        right = (my + 1) % N_DEV

        barrier_sem = pltpu.get_barrier_semaphore()
        for nbr in (left, right):
            pl.semaphore_signal(
                barrier_sem, inc=1,
                device_id=(nbr,), device_id_type=pl.DeviceIdType.MESH,
            )
        pl.semaphore_wait(barrier_sem, 2)

        partial = jnp.dot(a_ref[:, :], b_ref[:, :],
                          preferred_element_type=jnp.float32)
        comm_ref[0, :, :] = partial
        out_ref[:, :] = partial

        for h in range(N_DEV - 1):
            rdma = pltpu.make_async_remote_copy(
                src_ref=comm_ref.at[h],
                dst_ref=comm_ref.at[h + 1],
                send_sem=send_sems.at[h],
                recv_sem=recv_sems.at[h],
                device_id=(right,),
                device_id_type=pl.DeviceIdType.MESH,
            )
            rdma.start()
            rdma.wait()
            out_ref[:, :] += comm_ref[h + 1, :, :]

        z = out_ref[:, :]
        out_ref[:, :] = z * (1.0 / (1.0 + jnp.exp(-z)))

    return pl.pallas_call(
        body,
        out_shape=jax.ShapeDtypeStruct((m, n), jnp.float32),
        in_specs=[
            pl.BlockSpec(memory_space=pltpu.VMEM),
            pl.BlockSpec(memory_space=pltpu.VMEM),
        ],
        out_specs=pl.BlockSpec(memory_space=pltpu.VMEM),
        scratch_shapes=[
            pltpu.VMEM((N_DEV, m, n), jnp.float32),
            pltpu.SemaphoreType.DMA((N_DEV - 1,)),
            pltpu.SemaphoreType.DMA((N_DEV - 1,)),
        ],
        compiler_params=pltpu.CompilerParams(
            collective_id=0, vmem_limit_bytes=100 * 1024 * 1024
        ),
    )(A, B)


# device time: 104910 ns/iter; 3.1975x vs baseline; 3.1975x over previous
import jax
import jax.numpy as jnp
from jax import lax
from jax.experimental import pallas as pl
from jax.experimental.pallas import tpu as pltpu

N_DEV = 4


def kernel(A, B):
    m, k = A.shape
    k2, n = B.shape
    half = m // 2
    quar = m // 4
    W = n // 2

    def body(a_ref, b_ref, out_ref, rb1, rb2, send_sems, recv_sems):
        my = lax.axis_index("i")
        p1 = my ^ 1
        p2 = 3 - my

        barrier_sem = pltpu.get_barrier_semaphore()
        for nbr in (p1, p2):
            pl.semaphore_signal(
                barrier_sem, inc=1,
                device_id=(nbr,), device_id_type=pl.DeviceIdType.MESH,
            )
        pl.semaphore_wait(barrier_sem, 2)

        out_ref[:, :] = jnp.dot(a_ref[:, :], b_ref[:, :],
                                preferred_element_type=jnp.float32)

        g = (my ^ (my >> 1)) & 1
        t = (my >> 1) & 1
        u = my & 1
        streams = [
            dict(c0=0, pa=p1, pb=p2, h=g, q=t),
            dict(c0=W, pa=p2, pb=p1, h=t, q=u),
        ]

        def exchange(src, dst, s, ph, partner):
            r = pltpu.make_async_remote_copy(
                src_ref=src, dst_ref=dst,
                send_sem=send_sems.at[s, ph],
                recv_sem=recv_sems.at[s, ph],
                device_id=(partner,),
                device_id_type=pl.DeviceIdType.MESH,
            )
            r.start()
            return r

        rd = []
        for s, st in enumerate(streams):
            src = out_ref.at[pl.ds((1 - st["h"]) * half, half),
                             pl.ds(st["c0"], W)]
            rd.append(exchange(src, rb1.at[s], s, 0, st["pa"]))
        for s, st in enumerate(streams):
            rd[s].wait()
            rows, cols = pl.ds(st["h"] * half, half), pl.ds(st["c0"], W)
            out_ref[rows, cols] = out_ref[rows, cols] + rb1[s]

        rd = []
        for s, st in enumerate(streams):
            src = out_ref.at[
                pl.ds(st["h"] * half + (1 - st["q"]) * quar, quar),
                pl.ds(st["c0"], W)]
            rd.append(exchange(src, rb2.at[s], s, 1, st["pb"]))
        for s, st in enumerate(streams):
            rd[s].wait()
            rows = pl.ds(st["h"] * half + st["q"] * quar, quar)
            cols = pl.ds(st["c0"], W)
            z = out_ref[rows, cols] + rb2[s]
            out_ref[rows, cols] = z * (1.0 / (1.0 + jnp.exp(-z)))

        rd = []
        for s, st in enumerate(streams):
            blk = out_ref.at[pl.ds(st["h"] * half + st["q"] * quar, quar),
                             pl.ds(st["c0"], W)]
            rd.append(exchange(blk, blk, s, 2, st["pb"]))
        for r in rd:
            r.wait()

        rd = []
        for s, st in enumerate(streams):
            blk = out_ref.at[pl.ds(st["h"] * half, half),
                             pl.ds(st["c0"], W)]
            rd.append(exchange(blk, blk, s, 3, st["pa"]))
        for r in rd:
            r.wait()

    return pl.pallas_call(
        body,
        out_shape=jax.ShapeDtypeStruct((m, n), jnp.float32),
        in_specs=[
            pl.BlockSpec(memory_space=pltpu.VMEM),
            pl.BlockSpec(memory_space=pltpu.VMEM),
        ],
        out_specs=pl.BlockSpec(memory_space=pltpu.VMEM),
        scratch_shapes=[
            pltpu.VMEM((2, half, W), jnp.float32),
            pltpu.VMEM((2, quar, W), jnp.float32),
            pltpu.SemaphoreType.DMA((2, 4)),
            pltpu.SemaphoreType.DMA((2, 4)),
        ],
        compiler_params=pltpu.CompilerParams(
            collective_id=0, vmem_limit_bytes=100 * 1024 * 1024
        ),
    )(A, B)


# device time: 103519 ns/iter; 3.2404x vs baseline; 1.0134x over previous
import jax
import jax.numpy as jnp
from jax import lax
from jax.experimental import pallas as pl
from jax.experimental.pallas import tpu as pltpu

N_DEV = 4


def kernel(A, B):
    m, k = A.shape
    k2, n = B.shape
    half = m // 2
    quar = m // 4
    W = n // 2

    def body(a_ref, b_ref, out_ref, rb1, rb2, send_sems, recv_sems):
        my = lax.axis_index("i")
        p1 = my ^ 1
        p2 = 3 - my

        barrier_sem = pltpu.get_barrier_semaphore()
        for nbr in (p1, p2):
            pl.semaphore_signal(
                barrier_sem, inc=1,
                device_id=(nbr,), device_id_type=pl.DeviceIdType.MESH,
            )
        pl.semaphore_wait(barrier_sem, 2)

        g = (my ^ (my >> 1)) & 1
        t = (my >> 1) & 1
        u = my & 1
        streams = [
            dict(c0=0, pa=p1, pb=p2, h=g, q=t),
            dict(c0=W, pa=p2, pb=p1, h=t, q=u),
        ]

        def exchange(src, dst, s, ph, partner):
            r = pltpu.make_async_remote_copy(
                src_ref=src, dst_ref=dst,
                send_sem=send_sems.at[s, ph],
                recv_sem=recv_sems.at[s, ph],
                device_id=(partner,),
                device_id_type=pl.DeviceIdType.MESH,
            )
            r.start()
            return r

        def mm_block(r0, c0):
            out_ref[pl.ds(r0, half), pl.ds(c0, W)] = jnp.dot(
                a_ref[pl.ds(r0, half), :], b_ref[:, pl.ds(c0, W)],
                preferred_element_type=jnp.float32)

        rd = []
        for s, st in enumerate(streams):
            r0 = (1 - st["h"]) * half
            mm_block(r0, st["c0"])
            src = out_ref.at[pl.ds(r0, half), pl.ds(st["c0"], W)]
            rd.append(exchange(src, rb1.at[s], s, 0, st["pa"]))
        for s, st in enumerate(streams):
            mm_block(st["h"] * half, st["c0"])
        for s, st in enumerate(streams):
            rd[s].wait()
            rows, cols = pl.ds(st["h"] * half, half), pl.ds(st["c0"], W)
            out_ref[rows, cols] = out_ref[rows, cols] + rb1[s]

        rd = []
        for s, st in enumerate(streams):
            src = out_ref.at[
                pl.ds(st["h"] * half + (1 - st["q"]) * quar, quar),
                pl.ds(st["c0"], W)]
            rd.append(exchange(src, rb2.at[s], s, 1, st["pb"]))
        for s, st in enumerate(streams):
            rd[s].wait()
            rows = pl.ds(st["h"] * half + st["q"] * quar, quar)
            cols = pl.ds(st["c0"], W)
            z = out_ref[rows, cols] + rb2[s]
            out_ref[rows, cols] = z * (1.0 / (1.0 + jnp.exp(-z)))

        rd = []
        for s, st in enumerate(streams):
            blk = out_ref.at[pl.ds(st["h"] * half + st["q"] * quar, quar),
                             pl.ds(st["c0"], W)]
            rd.append(exchange(blk, blk, s, 2, st["pb"]))
        for r in rd:
            r.wait()

        rd = []
        for s, st in enumerate(streams):
            blk = out_ref.at[pl.ds(st["h"] * half, half),
                             pl.ds(st["c0"], W)]
            rd.append(exchange(blk, blk, s, 3, st["pa"]))
        for r in rd:
            r.wait()

    return pl.pallas_call(
        body,
        out_shape=jax.ShapeDtypeStruct((m, n), jnp.float32),
        in_specs=[
            pl.BlockSpec(memory_space=pltpu.VMEM),
            pl.BlockSpec(memory_space=pltpu.VMEM),
        ],
        out_specs=pl.BlockSpec(memory_space=pltpu.VMEM),
        scratch_shapes=[
            pltpu.VMEM((2, half, W), jnp.float32),
            pltpu.VMEM((2, quar, W), jnp.float32),
            pltpu.SemaphoreType.DMA((2, 4)),
            pltpu.SemaphoreType.DMA((2, 4)),
        ],
        compiler_params=pltpu.CompilerParams(
            collective_id=0, vmem_limit_bytes=100 * 1024 * 1024
        ),
    )(A, B)


# device time: 65302 ns/iter; 5.1369x vs baseline; 1.5852x over previous
import jax
import jax.numpy as jnp
from jax import lax
from jax.experimental import pallas as pl
from jax.experimental.pallas import tpu as pltpu

N_DEV = 4


def kernel(A, B):
    m, k = A.shape
    k2, n = B.shape
    half = m // 2
    quar = m // 4
    W = n // 2

    def body(a_ref, b_ref, out_ref, sb1, rb1, sb2, rb2, agb, agb2,
             send_sems, recv_sems):
        my = lax.axis_index("i")
        p1 = my ^ 1
        p2 = 3 - my

        barrier_sem = pltpu.get_barrier_semaphore()
        for nbr in (p1, p2):
            pl.semaphore_signal(
                barrier_sem, inc=1,
                device_id=(nbr,), device_id_type=pl.DeviceIdType.MESH,
            )
        pl.semaphore_wait(barrier_sem, 2)

        g = (my ^ (my >> 1)) & 1
        t = (my >> 1) & 1
        u = my & 1
        streams = [
            dict(c0=0, pa=p1, pb=p2, h=g, q=t),
            dict(c0=W, pa=p2, pb=p1, h=t, q=u),
        ]

        def exchange(src, dst, s, ph, partner):
            r = pltpu.make_async_remote_copy(
                src_ref=src, dst_ref=dst,
                send_sem=send_sems.at[s, ph],
                recv_sem=recv_sems.at[s, ph],
                device_id=(partner,),
                device_id_type=pl.DeviceIdType.MESH,
            )
            r.start()
            return r

        def mm(r0, c0):
            return jnp.dot(a_ref[pl.ds(r0, half), :],
                           b_ref[:, pl.ds(c0, W)],
                           preferred_element_type=jnp.float32)

        rd = []
        for s, st in enumerate(streams):
            sb1[s, :, :] = mm((1 - st["h"]) * half, st["c0"]).astype(
                jnp.bfloat16)
            rd.append(exchange(sb1.at[s], rb1.at[s], s, 0, st["pa"]))
        for s, st in enumerate(streams):
            out_ref[pl.ds(st["h"] * half, half), pl.ds(st["c0"], W)] = mm(
                st["h"] * half, st["c0"])

        rd2 = []
        for s, st in enumerate(streams):
            rd[s].wait()
            cols = pl.ds(st["c0"], W)
            ra = pl.ds(st["h"] * half + (1 - st["q"]) * quar, quar)
            za = (out_ref[ra, cols]
                  + rb1[s, pl.ds((1 - st["q"]) * quar, quar), :].astype(
                      jnp.float32))
            sb2[s, :, :] = za.astype(jnp.bfloat16)
            rd2.append(exchange(sb2.at[s], rb2.at[s], s, 1, st["pb"]))
            rb_ = pl.ds(st["h"] * half + st["q"] * quar, quar)
            out_ref[rb_, cols] = (
                out_ref[rb_, cols]
                + rb1[s, pl.ds(st["q"] * quar, quar), :].astype(jnp.float32))

        rd3 = []
        for s, st in enumerate(streams):
            rd2[s].wait()
            cols = pl.ds(st["c0"], W)
            rb_ = pl.ds(st["h"] * half + st["q"] * quar, quar)
            z = out_ref[rb_, cols] + rb2[s].astype(jnp.float32)
            zs = z * (1.0 / (1.0 + jnp.exp(-z)))
            out_ref[rb_, cols] = zs
            agb[s, pl.ds(st["q"] * quar, quar), :] = zs.astype(jnp.bfloat16)
            blk = agb.at[s, pl.ds(st["q"] * quar, quar)]
            rd3.append(exchange(blk, blk, s, 2, st["pb"]))

        rd4 = []
        for s, st in enumerate(streams):
            rd3[s].wait()
            rd4.append(exchange(agb.at[s], agb2.at[s], s, 3, st["pa"]))
        for s, st in enumerate(streams):
            out_ref[pl.ds(st["h"] * half + (1 - st["q"]) * quar, quar),
                    pl.ds(st["c0"], W)] = agb[
                s, pl.ds((1 - st["q"]) * quar, quar), :].astype(jnp.float32)
        for s, st in enumerate(streams):
            rd4[s].wait()
            out_ref[pl.ds((1 - st["h"]) * half, half),
                    pl.ds(st["c0"], W)] = agb2[s].astype(jnp.float32)

    return pl.pallas_call(
        body,
        out_shape=jax.ShapeDtypeStruct((m, n), jnp.float32),
        in_specs=[
            pl.BlockSpec(memory_space=pltpu.VMEM),
            pl.BlockSpec(memory_space=pltpu.VMEM),
        ],
        out_specs=pl.BlockSpec(memory_space=pltpu.VMEM),
        scratch_shapes=[
            pltpu.VMEM((2, half, W), jnp.bfloat16),
            pltpu.VMEM((2, half, W), jnp.bfloat16),
            pltpu.VMEM((2, quar, W), jnp.bfloat16),
            pltpu.VMEM((2, quar, W), jnp.bfloat16),
            pltpu.VMEM((2, half, W), jnp.bfloat16),
            pltpu.VMEM((2, half, W), jnp.bfloat16),
            pltpu.SemaphoreType.DMA((2, 4)),
            pltpu.SemaphoreType.DMA((2, 4)),
        ],
        compiler_params=pltpu.CompilerParams(
            collective_id=0, vmem_limit_bytes=100 * 1024 * 1024
        ),
    )(A, B)
